# baseline (device time: 14171 ns/iter reference)
import jax
import jax.numpy as jnp
from jax import lax
from jax.experimental import pallas as pl
from jax.experimental.pallas import tpu as pltpu

N_DEV = 8
N_STAGES = 3
ROW_SPLIT = (48, 48, 48, 48, 40, 40, 40, 40, 40, 40, 40, 40)
N_CHUNKS = len(ROW_SPLIT)
DIAG_NO_COMM = False


def kernel(t):
    m, n = t.shape
    row_off = [sum(ROW_SPLIT[:c]) for c in range(N_CHUNKS)]

    def body(x_ref, out_ref, acc_ref, comm_ref, send_sems, recv_sems):
        i = lax.axis_index("i")

        px = i + 1 - 2 * (i % 2)
        base = (i // 4) * 4
        py = base + 3 - (i - base)
        pz = (i + 4) % N_DEV
        dims = [px, py, pz]

        if not DIAG_NO_COMM:
            barrier_sem = pltpu.get_barrier_semaphore()
            for p in dims:
                pl.semaphore_signal(
                    barrier_sem, inc=1,
                    device_id=(p,), device_id_type=pl.DeviceIdType.MESH,
                )
            pl.semaphore_wait(barrier_sem, 3)

        def make_rdma(c, s):
            r0, rc = row_off[c], ROW_SPLIT[c]
            src = x_ref if s == 0 else acc_ref
            return pltpu.make_async_remote_copy(
                src_ref=src.at[pl.ds(r0, rc), :],
                dst_ref=comm_ref.at[s, pl.ds(r0, rc), :],
                send_sem=send_sems.at[c, s],
                recv_sem=recv_sems.at[c, s],
                device_id=(dims[(s + c) % 3],),
                device_id_type=pl.DeviceIdType.MESH,
            )

        rdmas = [[None] * N_STAGES for _ in range(N_CHUNKS)]
        for c in range(N_CHUNKS):
            rdmas[c][0] = make_rdma(c, 0)
            if not DIAG_NO_COMM:
                rdmas[c][0].start()

        for s in range(N_STAGES):
            for c in range(N_CHUNKS):
                r0, rc = row_off[c], ROW_SPLIT[c]
                if not DIAG_NO_COMM:
                    rdmas[c][s].wait()
                prev = x_ref if s == 0 else acc_ref
                acc_ref[pl.ds(r0, rc), :] = (
                    prev[pl.ds(r0, rc), :] + comm_ref[s, pl.ds(r0, rc), :]
                )
                if s + 1 < N_STAGES:
                    rdmas[c][s + 1] = make_rdma(c, s + 1)
                    if not DIAG_NO_COMM:
                        rdmas[c][s + 1].start()
                else:
                    sv = acc_ref[pl.ds(r0, rc), :]
                    r = jnp.maximum(sv, 0.0)
                    out_ref[pl.ds(r0, rc), :] = (
                        jnp.tanh(sv) * sv * sv + r * r * r
                    )

    return pl.pallas_call(
        body,
        out_shape=jax.ShapeDtypeStruct((m, n), jnp.float32),
        in_specs=[pl.BlockSpec(memory_space=pltpu.VMEM)],
        out_specs=pl.BlockSpec(memory_space=pltpu.VMEM),
        scratch_shapes=[
            pltpu.VMEM((m, n), jnp.float32),
            pltpu.VMEM((N_STAGES, m, n), jnp.float32),
            pltpu.SemaphoreType.DMA((N_CHUNKS, N_STAGES)),
            pltpu.SemaphoreType.DMA((N_CHUNKS, N_STAGES)),
        ],
        compiler_params=(None if DIAG_NO_COMM else pltpu.CompilerParams(collective_id=0)),
    )(t)
